# baseline (speedup 1.0000x reference)
import functools
import math

import jax
import jax.numpy as jnp
from jax import lax
from jax.experimental import pallas as pl
from jax.experimental.pallas import tpu as pltpu
from jax.experimental.pallas import tpu_sc as plsc

_B, _S, _D = 16, 4096, 1024
_TARGET = 512
_DIFF = _S % _TARGET
_STEP = math.ceil((_S - _DIFF) / _TARGET)
_OFF = _DIFF // 2

_R = _B * _TARGET
_NW = 32
_RPW = _R // _NW
_CMAX = 56
_CHS = [8, 56, 56, 56, 56, 24]
assert sum(_CHS) == _RPW and all(c <= _CMAX and c % 8 == 0 for c in _CHS)
_OFFS = [sum(_CHS[:i]) for i in range(len(_CHS))]
_NCHUNK = len(_CHS)
_NBUF = 2

_mesh = plsc.VectorSubcoreMesh(core_axis_name="c", subcore_axis_name="s")


@functools.partial(
    pl.kernel,
    mesh=_mesh,
    out_type=jax.ShapeDtypeStruct((_R, _D), jnp.float32),
    scratch_types=[
        pltpu.VMEM((_RPW,), jnp.int32),
        pltpu.VMEM((_CMAX, _D), jnp.float32),
        pltpu.VMEM((_CMAX, _D), jnp.float32),
        pltpu.SemaphoreType.DMA,
        pltpu.SemaphoreType.DMA,
        pltpu.SemaphoreType.DMA,
        pltpu.SemaphoreType.DMA,
    ],
)
def _downsample(x_hbm, out_hbm, idx_v, rows0, rows1,
                gsem0, gsem1, ssem0, ssem1):
    wid = lax.axis_index("s") * 2 + lax.axis_index("c")
    base = wid * _RPW
    lane = lax.iota(jnp.int32, 16)

    def build_idx(j):
        idx_v[pl.ds(j * 16, 16)] = (lane + (base + j * 16)) * _STEP + _OFF

    bufs = (rows0, rows1)
    gsems = (gsem0, gsem1)
    ssems = (ssem0, ssem1)

    def gather(g):
        n = _CHS[g]
        return pltpu.async_copy(
            x_hbm.at[idx_v.at[pl.ds(_OFFS[g], n)]],
            bufs[g % _NBUF].at[pl.ds(0, n)], gsems[g % _NBUF])

    def scatter(g):
        n = _CHS[g]
        return pltpu.async_copy(
            bufs[g % _NBUF].at[pl.ds(0, n)],
            out_hbm.at[pl.ds(base + _OFFS[g], n)], ssems[g % _NBUF])

    gathers = [None] * _NCHUNK
    scatters = [None] * _NCHUNK
    build_idx(0)
    for g in range(_NBUF - 1):
        gathers[g] = gather(g)
    for j in range(1, _RPW // 16):
        build_idx(j)
    for g in range(_NCHUNK):
        nxt = g + _NBUF - 1
        if nxt < _NCHUNK:
            if nxt >= _NBUF:
                scatters[nxt - _NBUF].wait()
            gathers[nxt] = gather(nxt)
        gathers[g].wait()
        scatters[g] = scatter(g)
    for g in range(_NCHUNK - _NBUF, _NCHUNK):
        scatters[g].wait()


def kernel(x):
    xf = x.reshape(_B * _S, _D)
    out = _downsample(xf)
    return out.reshape(_B, _TARGET, _D)


# speedup vs baseline: 1.0013x; 1.0013x over previous
import functools
import math

import jax
import jax.numpy as jnp
from jax import lax
from jax.experimental import pallas as pl
from jax.experimental.pallas import tpu as pltpu
from jax.experimental.pallas import tpu_sc as plsc

_B, _S, _D = 16, 4096, 1024
_TARGET = 512
_DIFF = _S % _TARGET
_STEP = math.ceil((_S - _DIFF) / _TARGET)
_OFF = _DIFF // 2

_R = _B * _TARGET
_NW = 32
_RPW = _R // _NW
_CMAX = 56
_CHS = [16, 48, 56, 56, 56, 24]
assert sum(_CHS) == _RPW and all(c <= _CMAX and c % 8 == 0 for c in _CHS)
_OFFS = [sum(_CHS[:i]) for i in range(len(_CHS))]
_NCHUNK = len(_CHS)
_NBUF = 2

_mesh = plsc.VectorSubcoreMesh(core_axis_name="c", subcore_axis_name="s")


@functools.partial(
    pl.kernel,
    mesh=_mesh,
    out_type=jax.ShapeDtypeStruct((_R, _D), jnp.float32),
    scratch_types=[
        pltpu.VMEM((_RPW,), jnp.int32),
        pltpu.VMEM((_CMAX, _D), jnp.float32),
        pltpu.VMEM((_CMAX, _D), jnp.float32),
        pltpu.SemaphoreType.DMA,
        pltpu.SemaphoreType.DMA,
        pltpu.SemaphoreType.DMA,
        pltpu.SemaphoreType.DMA,
    ],
)
def _downsample(x_hbm, out_hbm, idx_v, rows0, rows1,
                gsem0, gsem1, ssem0, ssem1):
    wid = lax.axis_index("s") * 2 + lax.axis_index("c")
    base = wid * _RPW
    lane = lax.iota(jnp.int32, 16)

    def build_idx(j):
        idx_v[pl.ds(j * 16, 16)] = (lane + (base + j * 16)) * _STEP + _OFF

    bufs = (rows0, rows1)
    gsems = (gsem0, gsem1)
    ssems = (ssem0, ssem1)

    def gather(g):
        n = _CHS[g]
        return pltpu.async_copy(
            x_hbm.at[idx_v.at[pl.ds(_OFFS[g], n)]],
            bufs[g % _NBUF].at[pl.ds(0, n)], gsems[g % _NBUF])

    def scatter(g):
        n = _CHS[g]
        return pltpu.async_copy(
            bufs[g % _NBUF].at[pl.ds(0, n)],
            out_hbm.at[pl.ds(base + _OFFS[g], n)], ssems[g % _NBUF])

    gathers = [None] * _NCHUNK
    scatters = [None] * _NCHUNK
    build_idx(0)
    for g in range(_NBUF - 1):
        gathers[g] = gather(g)
    for j in range(1, _RPW // 16):
        build_idx(j)
    for g in range(_NCHUNK):
        nxt = g + _NBUF - 1
        if nxt < _NCHUNK:
            if nxt >= _NBUF:
                scatters[nxt - _NBUF].wait()
            gathers[nxt] = gather(nxt)
        gathers[g].wait()
        scatters[g] = scatter(g)
    for g in range(_NCHUNK - _NBUF, _NCHUNK):
        scatters[g].wait()


def kernel(x):
    xf = x.reshape(_B * _S, _D)
    out = _downsample(xf)
    return out.reshape(_B, _TARGET, _D)


# speedup vs baseline: 1.0051x; 1.0038x over previous
import functools
import math

import jax
import jax.numpy as jnp
from jax import lax
from jax.experimental import pallas as pl
from jax.experimental.pallas import tpu as pltpu
from jax.experimental.pallas import tpu_sc as plsc

_B, _S, _D = 16, 4096, 1024
_TARGET = 512
_DIFF = _S % _TARGET
_STEP = math.ceil((_S - _DIFF) / _TARGET)
_OFF = _DIFF // 2

_R = _B * _TARGET
_NW = 32
_RPW = _R // _NW
_CMAX = 56
_CHS = [16, 56, 56, 56, 56, 16]
assert sum(_CHS) == _RPW and all(c <= _CMAX and c % 8 == 0 for c in _CHS)
_OFFS = [sum(_CHS[:i]) for i in range(len(_CHS))]
_NCHUNK = len(_CHS)
_NBUF = 2

_mesh = plsc.VectorSubcoreMesh(core_axis_name="c", subcore_axis_name="s")


@functools.partial(
    pl.kernel,
    mesh=_mesh,
    out_type=jax.ShapeDtypeStruct((_R, _D), jnp.float32),
    scratch_types=[
        pltpu.VMEM((_RPW,), jnp.int32),
        pltpu.VMEM((_CMAX, _D), jnp.float32),
        pltpu.VMEM((_CMAX, _D), jnp.float32),
        pltpu.SemaphoreType.DMA,
        pltpu.SemaphoreType.DMA,
        pltpu.SemaphoreType.DMA,
        pltpu.SemaphoreType.DMA,
    ],
)
def _downsample(x_hbm, out_hbm, idx_v, rows0, rows1,
                gsem0, gsem1, ssem0, ssem1):
    wid = lax.axis_index("s") * 2 + lax.axis_index("c")
    base = wid * _RPW
    lane = lax.iota(jnp.int32, 16)

    def build_idx(j):
        idx_v[pl.ds(j * 16, 16)] = (lane + (base + j * 16)) * _STEP + _OFF

    bufs = (rows0, rows1)
    gsems = (gsem0, gsem1)
    ssems = (ssem0, ssem1)

    def gather(g):
        n = _CHS[g]
        return pltpu.async_copy(
            x_hbm.at[idx_v.at[pl.ds(_OFFS[g], n)]],
            bufs[g % _NBUF].at[pl.ds(0, n)], gsems[g % _NBUF])

    def scatter(g):
        n = _CHS[g]
        return pltpu.async_copy(
            bufs[g % _NBUF].at[pl.ds(0, n)],
            out_hbm.at[pl.ds(base + _OFFS[g], n)], ssems[g % _NBUF])

    gathers = [None] * _NCHUNK
    scatters = [None] * _NCHUNK
    build_idx(0)
    for g in range(_NBUF - 1):
        gathers[g] = gather(g)
    for j in range(1, _RPW // 16):
        build_idx(j)
    for g in range(_NCHUNK):
        nxt = g + _NBUF - 1
        if nxt < _NCHUNK:
            if nxt >= _NBUF:
                scatters[nxt - _NBUF].wait()
            gathers[nxt] = gather(nxt)
        gathers[g].wait()
        scatters[g] = scatter(g)
    for g in range(_NCHUNK - _NBUF, _NCHUNK):
        scatters[g].wait()


def kernel(x):
    xf = x.reshape(_B * _S, _D)
    out = _downsample(xf)
    return out.reshape(_B, _TARGET, _D)
